# Initial kernel scaffold; baseline (speedup 1.0000x reference)
import jax
import jax.numpy as jnp
from jax.experimental import pallas as pl


def kernel(x, edge_index, edge_attr, W1, b1, W2, b2, W3, b3, W4, b4, We, be, v_gamma, v_beta, e_gamma, e_beta):
    raise NotImplementedError("write your pallas kernel here")


# speedup vs baseline: 1.6078x; 1.6078x over previous
import functools

import jax
import jax.numpy as jnp
from jax import lax
from jax.experimental import pallas as pl
from jax.experimental.pallas import tpu as pltpu
from jax.experimental.pallas import tpu_sc as plsc

NC = 2
NS = 16
NW = NC * NS
CA = 40
CB = 80
TW = 144


def _sigmoid(v):
    return 1.0 / (1.0 + jnp.exp(-v))


def _silu(v):
    return v * _sigmoid(v)


def _sca_body(t2, ea, srch, dsth, zeros, sums_out,
              src0, src1, dst0, dst1, ea0, ea1, g0, g1, ew0, ew1,
              table, sg0, sg1, se0, se1, ss0, ss1):
    n = zeros.shape[0]
    e = ea.shape[0]
    c = lax.axis_index("c")
    s = lax.axis_index("s")
    wid = c * NS + s
    rows_pt = n // NS
    src_v = (src0, src1)
    dst_v = (dst0, dst1)
    ea_v = (ea0, ea1)
    g_v = (g0, g1)
    ew_v = (ew0, ew1)
    sg = (sg0, sg1)
    se = (se0, se1)
    ss = (ss0, ss1)

    pltpu.sync_copy(zeros.at[pl.ds(s * rows_pt, rows_pt)],
                    table.at[pl.ds(s * rows_pt, rows_pt)])
    plsc.subcore_barrier()

    e_pw = e // NW
    nch = e_pw // CA
    base_w = wid * e_pw

    def prefetch(k, b):
        base = base_w + k * CA
        pltpu.sync_copy(srch.at[pl.ds(base, CA)], src_v[b])
        pltpu.sync_copy(dsth.at[pl.ds(base, CA)], dst_v[b])
        pltpu.async_copy(ea.at[pl.ds(base, CA)], ea_v[b], se[b])
        pltpu.async_copy(t2.at[dst_v[b]], g_v[b], sg[b])

    def compute(b):
        def row(r, rc):
            for j in range(8):
                a = ea_v[b][r, pl.ds(16 * j, 16)]
                w = 1.0 / (1.0 + jnp.exp(-a))
                ew_v[b][r, pl.ds(16 * j, 16)] = w * g_v[b][r, pl.ds(16 * j, 16)]
            iota_f = lax.iota(jnp.int32, 16).astype(jnp.float32)
            ew_v[b][r, pl.ds(128, 16)] = jnp.maximum(1.0 - iota_f, 0.0)
            return rc
        lax.fori_loop(0, CA, row, 0)

    def step(k, b, first):
        if first:
            prefetch(k + 1, 1 - b)
        else:
            @pl.when(k + 1 < nch)
            def _():
                pltpu.make_async_copy(ew_v[1 - b], table.at[src_v[1 - b]],
                                      ss[1 - b]).wait()
                prefetch(k + 1, 1 - b)
        pltpu.make_async_copy(ea.at[pl.ds(0, CA)], ea_v[b], se[b]).wait()
        pltpu.make_async_copy(t2.at[dst_v[b]], g_v[b], sg[b]).wait()
        compute(b)
        pltpu.async_copy(ew_v[b], table.at[src_v[b]], ss[b], add=True)

    prefetch(0, 0)
    step(0, 0, True)

    def pair(p, carry):
        step(2 * p + 1, 1, False)
        step(2 * p + 2, 0, False)
        return carry

    lax.fori_loop(0, (nch - 1) // 2, pair, 0)
    if nch % 2 == 0:
        step(nch - 1, 1, False)
    pltpu.make_async_copy(ew_v[0], table.at[src_v[0]], ss[0]).wait()
    pltpu.make_async_copy(ew_v[1], table.at[src_v[1]], ss[1]).wait()

    plsc.subcore_barrier()
    pltpu.sync_copy(table.at[pl.ds(s * rows_pt, rows_pt)],
                    sums_out.at[c, pl.ds(s * rows_pt, rows_pt)])


def _sc_scatter(t2, edge_attr, src, dst):
    n = t2.shape[0]
    e = edge_attr.shape[0]
    npad = ((n + 8 * NS - 1) // (8 * NS)) * (8 * NS)
    zeros = jnp.zeros((npad, TW), jnp.float32)
    fn = pl.kernel(
        _sca_body,
        out_type=jax.ShapeDtypeStruct((NC, npad, TW), jnp.float32),
        mesh=plsc.VectorSubcoreMesh(core_axis_name="c", subcore_axis_name="s",
                                    num_cores=NC, num_subcores=NS),
        scratch_types=[
            pltpu.VMEM((CA,), jnp.int32),
            pltpu.VMEM((CA,), jnp.int32),
            pltpu.VMEM((CA,), jnp.int32),
            pltpu.VMEM((CA,), jnp.int32),
            pltpu.VMEM((CA, 128), jnp.float32),
            pltpu.VMEM((CA, 128), jnp.float32),
            pltpu.VMEM((CA, 128), jnp.float32),
            pltpu.VMEM((CA, 128), jnp.float32),
            pltpu.VMEM((CA, TW), jnp.float32),
            pltpu.VMEM((CA, TW), jnp.float32),
            pltpu.VMEM_SHARED((npad, TW), jnp.float32),
            pltpu.SemaphoreType.DMA,
            pltpu.SemaphoreType.DMA,
            pltpu.SemaphoreType.DMA,
            pltpu.SemaphoreType.DMA,
            pltpu.SemaphoreType.DMA,
            pltpu.SemaphoreType.DMA,
        ],
        compiler_params=pltpu.CompilerParams(use_tc_tiling_on_sc=False),
    )
    return fn(t2, edge_attr, src, dst, zeros)


def _scb_body(t3, t4, srch, dsth, gsum_out,
              src0, src1, dst0, dst1, g30, g31, g40, g41, gs0, gs1,
              s30, s31, s40, s41, so0, so1):
    e = gsum_out.shape[0]
    c = lax.axis_index("c")
    s = lax.axis_index("s")
    wid = c * NS + s
    src_v = (src0, src1)
    dst_v = (dst0, dst1)
    g3_v = (g30, g31)
    g4_v = (g40, g41)
    gs_v = (gs0, gs1)
    s3 = (s30, s31)
    s4 = (s40, s41)
    so = (so0, so1)

    e_pw = e // NW
    nch = e_pw // CB
    base_w = wid * e_pw

    def prefetch(k, b):
        base = base_w + k * CB
        pltpu.sync_copy(srch.at[pl.ds(base, CB)], src_v[b])
        pltpu.sync_copy(dsth.at[pl.ds(base, CB)], dst_v[b])
        pltpu.async_copy(t3.at[src_v[b]], g3_v[b], s3[b])
        pltpu.async_copy(t4.at[dst_v[b]], g4_v[b], s4[b])

    def compute(b):
        def row(r, rc):
            for j in range(8):
                gs_v[b][r, pl.ds(16 * j, 16)] = (
                    g3_v[b][r, pl.ds(16 * j, 16)]
                    + g4_v[b][r, pl.ds(16 * j, 16)])
            return rc
        lax.fori_loop(0, CB, row, 0)

    def step(k, b, first):
        if first:
            prefetch(k + 1, 1 - b)
        else:
            @pl.when(k + 1 < nch)
            def _():
                pltpu.make_async_copy(gs_v[1 - b],
                                      gsum_out.at[pl.ds(0, CB)],
                                      so[1 - b]).wait()
                prefetch(k + 1, 1 - b)
        pltpu.make_async_copy(t3.at[src_v[b]], g3_v[b], s3[b]).wait()
        pltpu.make_async_copy(t4.at[dst_v[b]], g4_v[b], s4[b]).wait()
        compute(b)
        base = base_w + k * CB
        pltpu.async_copy(gs_v[b], gsum_out.at[pl.ds(base, CB)], so[b])

    prefetch(0, 0)
    step(0, 0, True)

    def pair(p, carry):
        step(2 * p + 1, 1, False)
        step(2 * p + 2, 0, False)
        return carry

    lax.fori_loop(0, (nch - 1) // 2, pair, 0)
    if nch % 2 == 0:
        step(nch - 1, 1, False)
    pltpu.make_async_copy(gs_v[0], gsum_out.at[pl.ds(0, CB)], so[0]).wait()
    pltpu.make_async_copy(gs_v[1], gsum_out.at[pl.ds(0, CB)], so[1]).wait()


def _sc_gsum(t3, t4, src, dst):
    e = src.shape[0]
    fn = pl.kernel(
        _scb_body,
        out_type=jax.ShapeDtypeStruct((e, 128), jnp.float32),
        mesh=plsc.VectorSubcoreMesh(core_axis_name="c", subcore_axis_name="s",
                                    num_cores=NC, num_subcores=NS),
        scratch_types=[
            pltpu.VMEM((CB,), jnp.int32),
            pltpu.VMEM((CB,), jnp.int32),
            pltpu.VMEM((CB,), jnp.int32),
            pltpu.VMEM((CB,), jnp.int32),
            pltpu.VMEM((CB, 128), jnp.float32),
            pltpu.VMEM((CB, 128), jnp.float32),
            pltpu.VMEM((CB, 128), jnp.float32),
            pltpu.VMEM((CB, 128), jnp.float32),
            pltpu.VMEM((CB, 128), jnp.float32),
            pltpu.VMEM((CB, 128), jnp.float32),
            pltpu.SemaphoreType.DMA,
            pltpu.SemaphoreType.DMA,
            pltpu.SemaphoreType.DMA,
            pltpu.SemaphoreType.DMA,
            pltpu.SemaphoreType.DMA,
            pltpu.SemaphoreType.DMA,
        ],
        compiler_params=pltpu.CompilerParams(use_tc_tiling_on_sc=False),
    )
    return fn(t3, t4, src, dst)


def _proj_body(x_ref, w_ref, b_ref, x1_ref, t2_ref, t3_ref, t4_ref):
    acc = jnp.dot(x_ref[...], w_ref[...],
                  preferred_element_type=jnp.float32) + b_ref[...]
    x1_ref[...] = acc[:, 0:128]
    t2_ref[...] = acc[:, 128:256]
    t3_ref[...] = acc[:, 256:384]
    t4_ref[...] = acc[:, 384:512]


def _projections(x, wcat, bcat):
    n, d = x.shape
    nb = 10
    rb = n // nb
    out = pl.pallas_call(
        _proj_body,
        grid=(nb,),
        in_specs=[
            pl.BlockSpec((rb, d), lambda i: (i, 0)),
            pl.BlockSpec((d, 4 * d), lambda i: (0, 0)),
            pl.BlockSpec((1, 4 * d), lambda i: (0, 0)),
        ],
        out_specs=[pl.BlockSpec((rb, d), lambda i: (i, 0))] * 4,
        out_shape=[jax.ShapeDtypeStruct((n, d), jnp.float32)] * 4,
    )(x, wcat, bcat)
    return out


def _xside_body(x_ref, x1_ref, sums_ref, vg_ref, vb_ref, out_ref):
    n = x_ref.shape[0]
    st = sums_ref[0, 0:n, :] + sums_ref[1, 0:n, :]
    cnt = st[:, 128:129]
    agg = st[:, 0:128] / jnp.maximum(cnt, 1.0)
    z = x1_ref[...] + agg
    mu = jnp.mean(z, axis=0, keepdims=True)
    var = jnp.mean((z - mu) * (z - mu), axis=0, keepdims=True)
    zn = vg_ref[...] * (z - mu) * lax.rsqrt(var + 1e-5) + vb_ref[...]
    out_ref[...] = x_ref[...] + _silu(zn)


def _xside(x, x1, sums, v_gamma, v_beta):
    n, d = x.shape
    return pl.pallas_call(
        _xside_body,
        out_shape=jax.ShapeDtypeStruct((n, d), jnp.float32),
    )(x, x1, sums, v_gamma.reshape(1, d), v_beta.reshape(1, d))


def _stats_body(ea_ref, gs_ref, we_ref, be_ref, acc_ref):
    i = pl.program_id(0)

    @pl.when(i == 0)
    def _():
        acc_ref[...] = jnp.zeros_like(acc_ref)

    z = (jnp.dot(ea_ref[...], we_ref[...], preferred_element_type=jnp.float32)
         + be_ref[...] + gs_ref[...])
    s1 = jnp.sum(z, axis=0, keepdims=True)
    s2 = jnp.sum(z * z, axis=0, keepdims=True)
    pad = jnp.zeros((6, s1.shape[1]), jnp.float32)
    acc_ref[...] += jnp.concatenate([s1, s2, pad], axis=0)


def _edge_stats(edge_attr, gsum, we, be, eb, nb):
    e, d = edge_attr.shape
    return pl.pallas_call(
        _stats_body,
        grid=(nb,),
        in_specs=[
            pl.BlockSpec((eb, d), lambda i: (i, 0)),
            pl.BlockSpec((eb, d), lambda i: (i, 0)),
            pl.BlockSpec((d, d), lambda i: (0, 0)),
            pl.BlockSpec((1, d), lambda i: (0, 0)),
        ],
        out_specs=pl.BlockSpec((8, d), lambda i: (0, 0)),
        out_shape=jax.ShapeDtypeStruct((8, d), jnp.float32),
    )(edge_attr, gsum, we, be)


def _efinal_body(ea_ref, gs_ref, we_ref, be_ref, acc_ref, eg_ref, eb_ref,
                 out_ref, *, inv_e):
    ea = ea_ref[...]
    z = (jnp.dot(ea, we_ref[...], preferred_element_type=jnp.float32)
         + be_ref[...] + gs_ref[...])
    mu = acc_ref[0:1, :] * inv_e
    var = acc_ref[1:2, :] * inv_e - mu * mu
    zn = eg_ref[...] * (z - mu) * lax.rsqrt(var + 1e-5) + eb_ref[...]
    out_ref[...] = ea + _silu(zn)


def _edge_final(edge_attr, gsum, we, be, acc, e_gamma, e_beta, eb, nb):
    e, d = edge_attr.shape
    return pl.pallas_call(
        functools.partial(_efinal_body, inv_e=1.0 / e),
        grid=(nb,),
        in_specs=[
            pl.BlockSpec((eb, d), lambda i: (i, 0)),
            pl.BlockSpec((eb, d), lambda i: (i, 0)),
            pl.BlockSpec((d, d), lambda i: (0, 0)),
            pl.BlockSpec((1, d), lambda i: (0, 0)),
            pl.BlockSpec((8, d), lambda i: (0, 0)),
            pl.BlockSpec((1, d), lambda i: (0, 0)),
            pl.BlockSpec((1, d), lambda i: (0, 0)),
        ],
        out_specs=pl.BlockSpec((eb, d), lambda i: (i, 0)),
        out_shape=jax.ShapeDtypeStruct((e, d), jnp.float32),
    )(edge_attr, gsum, we, be, acc, e_gamma.reshape(1, d),
      e_beta.reshape(1, d))


def kernel(x, edge_index, edge_attr, W1, b1, W2, b2, W3, b3, W4, b4,
           We, be, v_gamma, v_beta, e_gamma, e_beta):
    n, d = x.shape
    e = edge_attr.shape[0]
    src = edge_index[0]
    dst = edge_index[1]

    wcat = jnp.concatenate([W1, W2, W3, W4], axis=1)
    bcat = jnp.concatenate([b1, b2, b3, b4]).reshape(1, 4 * d)

    x1, t2, t3, t4 = _projections(x, wcat, bcat)
    sums = _sc_scatter(t2, edge_attr, src, dst)
    gsum = _sc_gsum(t3, t4, src, dst)

    x_out = _xside(x, x1, sums, v_gamma, v_beta)

    eb = 2000
    nb = e // eb
    be2 = be.reshape(1, d)
    acc = _edge_stats(edge_attr, gsum, We, be2, eb, nb)
    w_out = _edge_final(edge_attr, gsum, We, be2, acc, e_gamma, e_beta, eb, nb)
    return (x_out, w_out)


# speedup vs baseline: 5.9748x; 3.7161x over previous
import functools

import jax
import jax.numpy as jnp
from jax import lax
from jax.experimental import pallas as pl
from jax.experimental.pallas import tpu as pltpu
from jax.experimental.pallas import tpu_sc as plsc

NC = 2
NS = 16
NW = NC * NS
CA = 40
CB = 80
TW = 144


def _sigmoid(v):
    return 1.0 / (1.0 + jnp.exp(-v))


def _silu(v):
    return v * _sigmoid(v)


NPH = 2


def _sca_body(t2, ea, src3, dst3, zeros, sums_out,
              sidx, didx, ea0, ea1, g0, g1, ew0, ew1,
              table, sg0, sg1, se0, se1, ss0, ss1):
    n = zeros.shape[0]
    e = ea.shape[0]
    c = lax.axis_index("c")
    s = lax.axis_index("s")
    wid = c * NS + s
    rows_pt = n // NS
    ea_v = (ea0, ea1)
    g_v = (g0, g1)
    ew_v = (ew0, ew1)
    sg = (sg0, sg1)
    se = (se0, se1)
    ss = (ss0, ss1)

    pltpu.sync_copy(zeros.at[pl.ds(s * rows_pt, rows_pt)],
                    table.at[pl.ds(s * rows_pt, rows_pt)])
    plsc.subcore_barrier()

    e_pw = e // NW
    nch = e_pw // CA
    nchp = nch // NPH
    base_w = wid * e_pw

    def compute(b):
        def row(r, rc):
            for j in range(8):
                a = ea_v[b][r, pl.ds(16 * j, 16)]
                w = 1.0 / (1.0 + jnp.exp(-a))
                ew_v[b][r, pl.ds(16 * j, 16)] = w * g_v[b][r, pl.ds(16 * j, 16)]
            return rc
        lax.fori_loop(0, CA, row, 0)

    for ph in range(NPH):
        pbase = base_w + ph * nchp * CA
        pltpu.sync_copy(src3.at[wid, pl.ds(ph * nchp, nchp)], sidx)
        pltpu.sync_copy(dst3.at[wid, pl.ds(ph * nchp, nchp)], didx)

        def prefetch(k, b):
            pltpu.async_copy(ea.at[pl.ds(pbase + k * CA, CA)], ea_v[b], se[b])
            pltpu.async_copy(t2.at[didx.at[k]], g_v[b], sg[b])

        def step(k, b):
            @pl.when(k + 1 < nchp)
            def _():
                prefetch(k + 1, 1 - b)
            pltpu.make_async_copy(ea.at[pl.ds(0, CA)], ea_v[b], se[b]).wait()
            pltpu.make_async_copy(t2.at[didx.at[0]], g_v[b], sg[b]).wait()

            @pl.when(k >= 2)
            def _():
                pltpu.make_async_copy(ew_v[b], table.at[sidx.at[0]],
                                      ss[b]).wait()
            compute(b)
            pltpu.async_copy(ew_v[b], table.at[sidx.at[k]], ss[b], add=True)

        prefetch(0, 0)
        step(0, 0)

        def pair(p, carry):
            step(2 * p + 1, 1)
            step(2 * p + 2, 0)
            return carry

        lax.fori_loop(0, (nchp - 1) // 2, pair, 0)
        if nchp % 2 == 0:
            step(nchp - 1, 1)
        pltpu.make_async_copy(ew_v[0], table.at[sidx.at[0]], ss[0]).wait()
        pltpu.make_async_copy(ew_v[1], table.at[sidx.at[0]], ss[1]).wait()

    plsc.subcore_barrier()
    pltpu.sync_copy(table.at[pl.ds(s * rows_pt, rows_pt)],
                    sums_out.at[c, pl.ds(s * rows_pt, rows_pt)])


def _sc_scatter(t2, edge_attr, src, dst):
    n = t2.shape[0]
    e = edge_attr.shape[0]
    npad = ((n + 8 * NS - 1) // (8 * NS)) * (8 * NS)
    e_pw = e // NW
    nch = e_pw // CA
    nchp = nch // NPH
    src3 = src.reshape(NW, nch, CA)
    dst3 = dst.reshape(NW, nch, CA)
    zeros = jnp.zeros((npad, 128), jnp.float32)
    fn = pl.kernel(
        _sca_body,
        out_type=jax.ShapeDtypeStruct((NC, npad, 128), jnp.float32),
        mesh=plsc.VectorSubcoreMesh(core_axis_name="c", subcore_axis_name="s",
                                    num_cores=NC, num_subcores=NS),
        scratch_types=[
            pltpu.VMEM((nchp, CA), jnp.int32),
            pltpu.VMEM((nchp, CA), jnp.int32),
            pltpu.VMEM((CA, 128), jnp.float32),
            pltpu.VMEM((CA, 128), jnp.float32),
            pltpu.VMEM((CA, 128), jnp.float32),
            pltpu.VMEM((CA, 128), jnp.float32),
            pltpu.VMEM((CA, 128), jnp.float32),
            pltpu.VMEM((CA, 128), jnp.float32),
            pltpu.VMEM_SHARED((npad, 128), jnp.float32),
            pltpu.SemaphoreType.DMA,
            pltpu.SemaphoreType.DMA,
            pltpu.SemaphoreType.DMA,
            pltpu.SemaphoreType.DMA,
            pltpu.SemaphoreType.DMA,
            pltpu.SemaphoreType.DMA,
        ],
        compiler_params=pltpu.CompilerParams(use_tc_tiling_on_sc=False),
    )
    return fn(t2, edge_attr, src3, dst3, zeros)


def _scb_body(t3, t4, src3, dst3, zeros16, gsum_out, cnt_out,
              sidx, didx, g30, g31, g40, g41, gs0, gs1, ones_v,
              ctab, s30, s31, s40, s41, so0, so1, sc0, sc1):
    e = gsum_out.shape[0]
    n16 = zeros16.shape[0]
    c = lax.axis_index("c")
    s = lax.axis_index("s")
    wid = c * NS + s
    rows_pt = n16 // NS
    g3_v = (g30, g31)
    g4_v = (g40, g41)
    gs_v = (gs0, gs1)
    s3 = (s30, s31)
    s4 = (s40, s41)
    so = (so0, so1)
    sc = (sc0, sc1)

    pltpu.sync_copy(zeros16.at[pl.ds(s * rows_pt, rows_pt)],
                    ctab.at[pl.ds(s * rows_pt, rows_pt)])

    def fill(r, rc):
        ones_v[r, pl.ds(0, 16)] = jnp.full((16,), 1.0, jnp.float32)
        return rc
    lax.fori_loop(0, CB, fill, 0)
    plsc.subcore_barrier()

    e_pw = e // NW
    nch = e_pw // CB
    base_w = wid * e_pw
    pltpu.sync_copy(src3.at[wid], sidx)
    pltpu.sync_copy(dst3.at[wid], didx)

    def prefetch(k, b):
        pltpu.async_copy(t3.at[sidx.at[k]], g3_v[b], s3[b])
        pltpu.async_copy(t4.at[didx.at[k]], g4_v[b], s4[b])

    def compute(b):
        def row(r, rc):
            for j in range(8):
                gs_v[b][r, pl.ds(16 * j, 16)] = (
                    g3_v[b][r, pl.ds(16 * j, 16)]
                    + g4_v[b][r, pl.ds(16 * j, 16)])
            return rc
        lax.fori_loop(0, CB, row, 0)

    def step(k, b):
        @pl.when(k + 1 < nch)
        def _():
            prefetch(k + 1, 1 - b)
        pltpu.make_async_copy(t3.at[sidx.at[0]], g3_v[b], s3[b]).wait()
        pltpu.make_async_copy(t4.at[didx.at[0]], g4_v[b], s4[b]).wait()

        @pl.when(k >= 2)
        def _():
            pltpu.make_async_copy(gs_v[b], gsum_out.at[pl.ds(0, CB)],
                                  so[b]).wait()
            pltpu.make_async_copy(ones_v, ctab.at[sidx.at[0]], sc[b]).wait()
        compute(b)
        pltpu.async_copy(gs_v[b], gsum_out.at[pl.ds(base_w + k * CB, CB)],
                         so[b])
        pltpu.async_copy(ones_v, ctab.at[sidx.at[k]], sc[b], add=True)

    prefetch(0, 0)
    step(0, 0)

    def pair(p, carry):
        step(2 * p + 1, 1)
        step(2 * p + 2, 0)
        return carry

    lax.fori_loop(0, (nch - 1) // 2, pair, 0)
    if nch % 2 == 0:
        step(nch - 1, 1)
    for b in (0, 1):
        pltpu.make_async_copy(gs_v[b], gsum_out.at[pl.ds(0, CB)], so[b]).wait()
        pltpu.make_async_copy(ones_v, ctab.at[sidx.at[0]], sc[b]).wait()

    plsc.subcore_barrier()
    pltpu.sync_copy(ctab.at[pl.ds(s * rows_pt, rows_pt)],
                    cnt_out.at[c, pl.ds(s * rows_pt, rows_pt)])


def _sc_gsum(t3, t4, src, dst):
    e = src.shape[0]
    n = t3.shape[0]
    npad = ((n + 8 * NS - 1) // (8 * NS)) * (8 * NS)
    e_pw = e // NW
    nch = e_pw // CB
    src3 = src.reshape(NW, nch, CB)
    dst3 = dst.reshape(NW, nch, CB)
    zeros16 = jnp.zeros((npad, 16), jnp.float32)
    fn = pl.kernel(
        _scb_body,
        out_type=(jax.ShapeDtypeStruct((e, 128), jnp.float32),
                  jax.ShapeDtypeStruct((NC, npad, 16), jnp.float32)),
        mesh=plsc.VectorSubcoreMesh(core_axis_name="c", subcore_axis_name="s",
                                    num_cores=NC, num_subcores=NS),
        scratch_types=[
            pltpu.VMEM((nch, CB), jnp.int32),
            pltpu.VMEM((nch, CB), jnp.int32),
            pltpu.VMEM((CB, 128), jnp.float32),
            pltpu.VMEM((CB, 128), jnp.float32),
            pltpu.VMEM((CB, 128), jnp.float32),
            pltpu.VMEM((CB, 128), jnp.float32),
            pltpu.VMEM((CB, 128), jnp.float32),
            pltpu.VMEM((CB, 128), jnp.float32),
            pltpu.VMEM((CB, 16), jnp.float32),
            pltpu.VMEM_SHARED((npad, 16), jnp.float32),
            pltpu.SemaphoreType.DMA,
            pltpu.SemaphoreType.DMA,
            pltpu.SemaphoreType.DMA,
            pltpu.SemaphoreType.DMA,
            pltpu.SemaphoreType.DMA,
            pltpu.SemaphoreType.DMA,
            pltpu.SemaphoreType.DMA,
            pltpu.SemaphoreType.DMA,
        ],
        compiler_params=pltpu.CompilerParams(use_tc_tiling_on_sc=False),
    )
    return fn(t3, t4, src3, dst3, zeros16)


def _proj_body(x_ref, w_ref, b_ref, x1_ref, t2_ref, t3_ref, t4_ref):
    acc = jnp.dot(x_ref[...], w_ref[...],
                  preferred_element_type=jnp.float32) + b_ref[...]
    x1_ref[...] = acc[:, 0:128]
    t2_ref[...] = acc[:, 128:256]
    t3_ref[...] = acc[:, 256:384]
    t4_ref[...] = acc[:, 384:512]


def _projections(x, wcat, bcat):
    n, d = x.shape
    nb = 10
    rb = n // nb
    out = pl.pallas_call(
        _proj_body,
        grid=(nb,),
        in_specs=[
            pl.BlockSpec((rb, d), lambda i: (i, 0)),
            pl.BlockSpec((d, 4 * d), lambda i: (0, 0)),
            pl.BlockSpec((1, 4 * d), lambda i: (0, 0)),
        ],
        out_specs=[pl.BlockSpec((rb, d), lambda i: (i, 0))] * 4,
        out_shape=[jax.ShapeDtypeStruct((n, d), jnp.float32)] * 4,
    )(x, wcat, bcat)
    return out


def _xside_body(x_ref, x1_ref, sums_ref, cnts_ref, vg_ref, vb_ref, out_ref):
    n = x_ref.shape[0]
    st = sums_ref[0, 0:n, :] + sums_ref[1, 0:n, :]
    cnt = cnts_ref[0, 0:n, 0:1] + cnts_ref[1, 0:n, 0:1]
    agg = st / jnp.maximum(cnt, 1.0)
    z = x1_ref[...] + agg
    mu = jnp.mean(z, axis=0, keepdims=True)
    var = jnp.mean((z - mu) * (z - mu), axis=0, keepdims=True)
    zn = vg_ref[...] * (z - mu) * lax.rsqrt(var + 1e-5) + vb_ref[...]
    out_ref[...] = x_ref[...] + _silu(zn)


def _xside(x, x1, sums, cnts, v_gamma, v_beta):
    n, d = x.shape
    return pl.pallas_call(
        _xside_body,
        out_shape=jax.ShapeDtypeStruct((n, d), jnp.float32),
    )(x, x1, sums, cnts, v_gamma.reshape(1, d), v_beta.reshape(1, d))


def _stats_body(ea_ref, gs_ref, we_ref, be_ref, acc_ref):
    i = pl.program_id(0)

    @pl.when(i == 0)
    def _():
        acc_ref[...] = jnp.zeros_like(acc_ref)

    z = (jnp.dot(ea_ref[...], we_ref[...], preferred_element_type=jnp.float32)
         + be_ref[...] + gs_ref[...])
    s1 = jnp.sum(z, axis=0, keepdims=True)
    s2 = jnp.sum(z * z, axis=0, keepdims=True)
    pad = jnp.zeros((6, s1.shape[1]), jnp.float32)
    acc_ref[...] += jnp.concatenate([s1, s2, pad], axis=0)


def _edge_stats(edge_attr, gsum, we, be, eb, nb):
    e, d = edge_attr.shape
    return pl.pallas_call(
        _stats_body,
        grid=(nb,),
        in_specs=[
            pl.BlockSpec((eb, d), lambda i: (i, 0)),
            pl.BlockSpec((eb, d), lambda i: (i, 0)),
            pl.BlockSpec((d, d), lambda i: (0, 0)),
            pl.BlockSpec((1, d), lambda i: (0, 0)),
        ],
        out_specs=pl.BlockSpec((8, d), lambda i: (0, 0)),
        out_shape=jax.ShapeDtypeStruct((8, d), jnp.float32),
    )(edge_attr, gsum, we, be)


def _efinal_body(ea_ref, gs_ref, we_ref, be_ref, acc_ref, eg_ref, eb_ref,
                 out_ref, *, inv_e):
    ea = ea_ref[...]
    z = (jnp.dot(ea, we_ref[...], preferred_element_type=jnp.float32)
         + be_ref[...] + gs_ref[...])
    mu = acc_ref[0:1, :] * inv_e
    var = acc_ref[1:2, :] * inv_e - mu * mu
    zn = eg_ref[...] * (z - mu) * lax.rsqrt(var + 1e-5) + eb_ref[...]
    out_ref[...] = ea + _silu(zn)


def _edge_final(edge_attr, gsum, we, be, acc, e_gamma, e_beta, eb, nb):
    e, d = edge_attr.shape
    return pl.pallas_call(
        functools.partial(_efinal_body, inv_e=1.0 / e),
        grid=(nb,),
        in_specs=[
            pl.BlockSpec((eb, d), lambda i: (i, 0)),
            pl.BlockSpec((eb, d), lambda i: (i, 0)),
            pl.BlockSpec((d, d), lambda i: (0, 0)),
            pl.BlockSpec((1, d), lambda i: (0, 0)),
            pl.BlockSpec((8, d), lambda i: (0, 0)),
            pl.BlockSpec((1, d), lambda i: (0, 0)),
            pl.BlockSpec((1, d), lambda i: (0, 0)),
        ],
        out_specs=pl.BlockSpec((eb, d), lambda i: (i, 0)),
        out_shape=jax.ShapeDtypeStruct((e, d), jnp.float32),
    )(edge_attr, gsum, we, be, acc, e_gamma.reshape(1, d),
      e_beta.reshape(1, d))


def kernel(x, edge_index, edge_attr, W1, b1, W2, b2, W3, b3, W4, b4,
           We, be, v_gamma, v_beta, e_gamma, e_beta):
    n, d = x.shape
    e = edge_attr.shape[0]
    src = edge_index[0]
    dst = edge_index[1]

    wcat = jnp.concatenate([W1, W2, W3, W4], axis=1)
    bcat = jnp.concatenate([b1, b2, b3, b4]).reshape(1, 4 * d)

    x1, t2, t3, t4 = _projections(x, wcat, bcat)
    sums = _sc_scatter(t2, edge_attr, src, dst)
    gsum, cnts = _sc_gsum(t3, t4, src, dst)

    x_out = _xside(x, x1, sums, cnts, v_gamma, v_beta)

    eb = 2000
    nb = e // eb
    be2 = be.reshape(1, d)
    acc = _edge_stats(edge_attr, gsum, We, be2, eb, nb)
    w_out = _edge_final(edge_attr, gsum, We, be2, acc, e_gamma, e_beta, eb, nb)
    return (x_out, w_out)


# speedup vs baseline: 6.5921x; 1.1033x over previous
import functools

import jax
import jax.numpy as jnp
from jax import lax
from jax.experimental import pallas as pl
from jax.experimental.pallas import tpu as pltpu
from jax.experimental.pallas import tpu_sc as plsc

NC = 2
NS = 16
NW = NC * NS
CA = 40
CB = 80
TW = 144


def _sigmoid(v):
    return 1.0 / (1.0 + jnp.exp(-v))


def _silu(v):
    return v * _sigmoid(v)


NPH = 2


def _sca_body(t2, ea, src3, dst3, zeros, sums_out,
              sidx, didx, ea0, ea1, ea2, g0, g1, g2,
              table, sg0, sg1, sg2, se0, se1, se2, ss0, ss1, ss2):
    n = zeros.shape[0]
    e = ea.shape[0]
    c = lax.axis_index("c")
    s = lax.axis_index("s")
    wid = c * NS + s
    rows_pt = n // NS
    ea_v = (ea0, ea1, ea2)
    g_v = (g0, g1, g2)
    sg = (sg0, sg1, sg2)
    se = (se0, se1, se2)
    ss = (ss0, ss1, ss2)

    pltpu.sync_copy(zeros.at[pl.ds(s * rows_pt, rows_pt)],
                    table.at[pl.ds(s * rows_pt, rows_pt)])
    plsc.subcore_barrier()

    e_pw = e // NW
    nch = e_pw // CA
    nchp = nch // NPH
    base_w = wid * e_pw

    def compute(b):
        def row(r, rc):
            for j in range(8):
                a = ea_v[b][r, pl.ds(16 * j, 16)]
                w = 1.0 / (1.0 + jnp.exp(-a))
                g_v[b][r, pl.ds(16 * j, 16)] = w * g_v[b][r, pl.ds(16 * j, 16)]
            return rc
        lax.fori_loop(0, CA, row, 0)

    for ph in range(NPH):
        pbase = base_w + ph * nchp * CA
        pltpu.sync_copy(src3.at[wid, pl.ds(ph * nchp, nchp)], sidx)
        pltpu.sync_copy(dst3.at[wid, pl.ds(ph * nchp, nchp)], didx)

        def prefetch(k, b):
            pltpu.async_copy(ea.at[pl.ds(pbase + k * CA, CA)], ea_v[b], se[b])
            pltpu.async_copy(t2.at[didx.at[k]], g_v[b], sg[b])

        def step(k, b):
            bn = (b + 1) % 3

            @pl.when(k + 1 < nchp)
            def _():
                @pl.when(k >= 2)
                def _():
                    pltpu.make_async_copy(g_v[bn], table.at[sidx.at[0]],
                                          ss[bn]).wait()
                prefetch(k + 1, bn)
            pltpu.make_async_copy(ea.at[pl.ds(0, CA)], ea_v[b], se[b]).wait()
            pltpu.make_async_copy(t2.at[didx.at[0]], g_v[b], sg[b]).wait()
            compute(b)
            pltpu.async_copy(g_v[b], table.at[sidx.at[k]], ss[b], add=True)

        prefetch(0, 0)
        step(0, 0)
        step(1, 1)

        def triple(p, carry):
            step(3 * p + 2, 2)
            step(3 * p + 3, 0)
            step(3 * p + 4, 1)
            return carry

        ntr = (nchp - 2) // 3
        lax.fori_loop(0, ntr, triple, 0)
        for k in range(2 + 3 * ntr, nchp):
            step(k, k % 3)
        for b in range(3):
            pltpu.make_async_copy(g_v[b], table.at[sidx.at[0]], ss[b]).wait()

    plsc.subcore_barrier()
    pltpu.sync_copy(table.at[pl.ds(s * rows_pt, rows_pt)],
                    sums_out.at[c, pl.ds(s * rows_pt, rows_pt)])


def _sc_scatter(t2, edge_attr, src, dst):
    n = t2.shape[0]
    e = edge_attr.shape[0]
    npad = ((n + 8 * NS - 1) // (8 * NS)) * (8 * NS)
    e_pw = e // NW
    nch = e_pw // CA
    nchp = nch // NPH
    src3 = src.reshape(NW, nch, CA)
    dst3 = dst.reshape(NW, nch, CA)
    zeros = jnp.zeros((npad, 128), jnp.float32)
    fn = pl.kernel(
        _sca_body,
        out_type=jax.ShapeDtypeStruct((NC, npad, 128), jnp.float32),
        mesh=plsc.VectorSubcoreMesh(core_axis_name="c", subcore_axis_name="s",
                                    num_cores=NC, num_subcores=NS),
        scratch_types=[
            pltpu.VMEM((nchp, CA), jnp.int32),
            pltpu.VMEM((nchp, CA), jnp.int32),
            pltpu.VMEM((CA, 128), jnp.float32),
            pltpu.VMEM((CA, 128), jnp.float32),
            pltpu.VMEM((CA, 128), jnp.float32),
            pltpu.VMEM((CA, 128), jnp.float32),
            pltpu.VMEM((CA, 128), jnp.float32),
            pltpu.VMEM((CA, 128), jnp.float32),
            pltpu.VMEM_SHARED((npad, 128), jnp.float32),
            pltpu.SemaphoreType.DMA,
            pltpu.SemaphoreType.DMA,
            pltpu.SemaphoreType.DMA,
            pltpu.SemaphoreType.DMA,
            pltpu.SemaphoreType.DMA,
            pltpu.SemaphoreType.DMA,
            pltpu.SemaphoreType.DMA,
            pltpu.SemaphoreType.DMA,
            pltpu.SemaphoreType.DMA,
        ],
        compiler_params=pltpu.CompilerParams(use_tc_tiling_on_sc=False),
    )
    return fn(t2, edge_attr, src3, dst3, zeros)


def _scb_body(t3, t4, src3, dst3, zeros16, gsum_out, cnt_out,
              sidx, didx, g30, g31, g40, g41, gs0, gs1, ones_v,
              ctab, s30, s31, s40, s41, so0, so1, sc0, sc1):
    e = gsum_out.shape[0]
    n16 = zeros16.shape[0]
    c = lax.axis_index("c")
    s = lax.axis_index("s")
    wid = c * NS + s
    rows_pt = n16 // NS
    g3_v = (g30, g31)
    g4_v = (g40, g41)
    gs_v = (gs0, gs1)
    s3 = (s30, s31)
    s4 = (s40, s41)
    so = (so0, so1)
    sc = (sc0, sc1)

    pltpu.sync_copy(zeros16.at[pl.ds(s * rows_pt, rows_pt)],
                    ctab.at[pl.ds(s * rows_pt, rows_pt)])

    def fill(r, rc):
        ones_v[r, pl.ds(0, 16)] = jnp.full((16,), 1.0, jnp.float32)
        return rc
    lax.fori_loop(0, CB, fill, 0)
    plsc.subcore_barrier()

    e_pw = e // NW
    nch = e_pw // CB
    base_w = wid * e_pw
    pltpu.sync_copy(src3.at[wid], sidx)
    pltpu.sync_copy(dst3.at[wid], didx)

    def prefetch(k, b):
        pltpu.async_copy(t3.at[sidx.at[k]], g3_v[b], s3[b])
        pltpu.async_copy(t4.at[didx.at[k]], g4_v[b], s4[b])

    def compute(b):
        def row(r, rc):
            for j in range(8):
                gs_v[b][r, pl.ds(16 * j, 16)] = (
                    g3_v[b][r, pl.ds(16 * j, 16)]
                    + g4_v[b][r, pl.ds(16 * j, 16)])
            return rc
        lax.fori_loop(0, CB, row, 0)

    def step(k, b):
        @pl.when(k + 1 < nch)
        def _():
            prefetch(k + 1, 1 - b)
        pltpu.make_async_copy(t3.at[sidx.at[0]], g3_v[b], s3[b]).wait()
        pltpu.make_async_copy(t4.at[didx.at[0]], g4_v[b], s4[b]).wait()

        @pl.when(k >= 2)
        def _():
            pltpu.make_async_copy(gs_v[b], gsum_out.at[pl.ds(0, CB)],
                                  so[b]).wait()
            pltpu.make_async_copy(ones_v, ctab.at[sidx.at[0]], sc[b]).wait()
        compute(b)
        pltpu.async_copy(gs_v[b], gsum_out.at[pl.ds(base_w + k * CB, CB)],
                         so[b])
        pltpu.async_copy(ones_v, ctab.at[sidx.at[k]], sc[b], add=True)

    prefetch(0, 0)
    step(0, 0)

    def pair(p, carry):
        step(2 * p + 1, 1)
        step(2 * p + 2, 0)
        return carry

    lax.fori_loop(0, (nch - 1) // 2, pair, 0)
    if nch % 2 == 0:
        step(nch - 1, 1)
    for b in (0, 1):
        pltpu.make_async_copy(gs_v[b], gsum_out.at[pl.ds(0, CB)], so[b]).wait()
        pltpu.make_async_copy(ones_v, ctab.at[sidx.at[0]], sc[b]).wait()

    plsc.subcore_barrier()
    pltpu.sync_copy(ctab.at[pl.ds(s * rows_pt, rows_pt)],
                    cnt_out.at[c, pl.ds(s * rows_pt, rows_pt)])


def _sc_gsum(t3, t4, src, dst):
    e = src.shape[0]
    n = t3.shape[0]
    npad = ((n + 8 * NS - 1) // (8 * NS)) * (8 * NS)
    e_pw = e // NW
    nch = e_pw // CB
    src3 = src.reshape(NW, nch, CB)
    dst3 = dst.reshape(NW, nch, CB)
    zeros16 = jnp.zeros((npad, 16), jnp.float32)
    fn = pl.kernel(
        _scb_body,
        out_type=(jax.ShapeDtypeStruct((e, 128), jnp.float32),
                  jax.ShapeDtypeStruct((NC, npad, 16), jnp.float32)),
        mesh=plsc.VectorSubcoreMesh(core_axis_name="c", subcore_axis_name="s",
                                    num_cores=NC, num_subcores=NS),
        scratch_types=[
            pltpu.VMEM((nch, CB), jnp.int32),
            pltpu.VMEM((nch, CB), jnp.int32),
            pltpu.VMEM((CB, 128), jnp.float32),
            pltpu.VMEM((CB, 128), jnp.float32),
            pltpu.VMEM((CB, 128), jnp.float32),
            pltpu.VMEM((CB, 128), jnp.float32),
            pltpu.VMEM((CB, 128), jnp.float32),
            pltpu.VMEM((CB, 128), jnp.float32),
            pltpu.VMEM((CB, 16), jnp.float32),
            pltpu.VMEM_SHARED((npad, 16), jnp.float32),
            pltpu.SemaphoreType.DMA,
            pltpu.SemaphoreType.DMA,
            pltpu.SemaphoreType.DMA,
            pltpu.SemaphoreType.DMA,
            pltpu.SemaphoreType.DMA,
            pltpu.SemaphoreType.DMA,
            pltpu.SemaphoreType.DMA,
            pltpu.SemaphoreType.DMA,
        ],
        compiler_params=pltpu.CompilerParams(use_tc_tiling_on_sc=False),
    )
    return fn(t3, t4, src3, dst3, zeros16)


def _proj_body(x_ref, w_ref, b_ref, x1_ref, t2_ref, t3_ref, t4_ref):
    acc = jnp.dot(x_ref[...], w_ref[...],
                  preferred_element_type=jnp.float32) + b_ref[...]
    x1_ref[...] = acc[:, 0:128]
    t2_ref[...] = acc[:, 128:256]
    t3_ref[...] = acc[:, 256:384]
    t4_ref[...] = acc[:, 384:512]


def _projections(x, wcat, bcat):
    n, d = x.shape
    nb = 10
    rb = n // nb
    out = pl.pallas_call(
        _proj_body,
        grid=(nb,),
        in_specs=[
            pl.BlockSpec((rb, d), lambda i: (i, 0)),
            pl.BlockSpec((d, 4 * d), lambda i: (0, 0)),
            pl.BlockSpec((1, 4 * d), lambda i: (0, 0)),
        ],
        out_specs=[pl.BlockSpec((rb, d), lambda i: (i, 0))] * 4,
        out_shape=[jax.ShapeDtypeStruct((n, d), jnp.float32)] * 4,
    )(x, wcat, bcat)
    return out


def _xside_body(x_ref, x1_ref, sums_ref, cnts_ref, vg_ref, vb_ref, out_ref):
    n = x_ref.shape[0]
    st = sums_ref[0, 0:n, :] + sums_ref[1, 0:n, :]
    cnt = cnts_ref[0, 0:n, 0:1] + cnts_ref[1, 0:n, 0:1]
    agg = st / jnp.maximum(cnt, 1.0)
    z = x1_ref[...] + agg
    mu = jnp.mean(z, axis=0, keepdims=True)
    var = jnp.mean((z - mu) * (z - mu), axis=0, keepdims=True)
    zn = vg_ref[...] * (z - mu) * lax.rsqrt(var + 1e-5) + vb_ref[...]
    out_ref[...] = x_ref[...] + _silu(zn)


def _xside(x, x1, sums, cnts, v_gamma, v_beta):
    n, d = x.shape
    return pl.pallas_call(
        _xside_body,
        out_shape=jax.ShapeDtypeStruct((n, d), jnp.float32),
    )(x, x1, sums, cnts, v_gamma.reshape(1, d), v_beta.reshape(1, d))


def _stats_body(ea_ref, gs_ref, we_ref, be_ref, acc_ref):
    i = pl.program_id(0)

    @pl.when(i == 0)
    def _():
        acc_ref[...] = jnp.zeros_like(acc_ref)

    z = (jnp.dot(ea_ref[...], we_ref[...], preferred_element_type=jnp.float32)
         + be_ref[...] + gs_ref[...])
    s1 = jnp.sum(z, axis=0, keepdims=True)
    s2 = jnp.sum(z * z, axis=0, keepdims=True)
    pad = jnp.zeros((6, s1.shape[1]), jnp.float32)
    acc_ref[...] += jnp.concatenate([s1, s2, pad], axis=0)


def _edge_stats(edge_attr, gsum, we, be, eb, nb):
    e, d = edge_attr.shape
    return pl.pallas_call(
        _stats_body,
        grid=(nb,),
        in_specs=[
            pl.BlockSpec((eb, d), lambda i: (i, 0)),
            pl.BlockSpec((eb, d), lambda i: (i, 0)),
            pl.BlockSpec((d, d), lambda i: (0, 0)),
            pl.BlockSpec((1, d), lambda i: (0, 0)),
        ],
        out_specs=pl.BlockSpec((8, d), lambda i: (0, 0)),
        out_shape=jax.ShapeDtypeStruct((8, d), jnp.float32),
    )(edge_attr, gsum, we, be)


def _efinal_body(ea_ref, gs_ref, we_ref, be_ref, acc_ref, eg_ref, eb_ref,
                 out_ref, *, inv_e):
    ea = ea_ref[...]
    z = (jnp.dot(ea, we_ref[...], preferred_element_type=jnp.float32)
         + be_ref[...] + gs_ref[...])
    mu = acc_ref[0:1, :] * inv_e
    var = acc_ref[1:2, :] * inv_e - mu * mu
    zn = eg_ref[...] * (z - mu) * lax.rsqrt(var + 1e-5) + eb_ref[...]
    out_ref[...] = ea + _silu(zn)


def _edge_final(edge_attr, gsum, we, be, acc, e_gamma, e_beta, eb, nb):
    e, d = edge_attr.shape
    return pl.pallas_call(
        functools.partial(_efinal_body, inv_e=1.0 / e),
        grid=(nb,),
        in_specs=[
            pl.BlockSpec((eb, d), lambda i: (i, 0)),
            pl.BlockSpec((eb, d), lambda i: (i, 0)),
            pl.BlockSpec((d, d), lambda i: (0, 0)),
            pl.BlockSpec((1, d), lambda i: (0, 0)),
            pl.BlockSpec((8, d), lambda i: (0, 0)),
            pl.BlockSpec((1, d), lambda i: (0, 0)),
            pl.BlockSpec((1, d), lambda i: (0, 0)),
        ],
        out_specs=pl.BlockSpec((eb, d), lambda i: (i, 0)),
        out_shape=jax.ShapeDtypeStruct((e, d), jnp.float32),
    )(edge_attr, gsum, we, be, acc, e_gamma.reshape(1, d),
      e_beta.reshape(1, d))


def kernel(x, edge_index, edge_attr, W1, b1, W2, b2, W3, b3, W4, b4,
           We, be, v_gamma, v_beta, e_gamma, e_beta):
    n, d = x.shape
    e = edge_attr.shape[0]
    src = edge_index[0]
    dst = edge_index[1]

    wcat = jnp.concatenate([W1, W2, W3, W4], axis=1)
    bcat = jnp.concatenate([b1, b2, b3, b4]).reshape(1, 4 * d)

    x1, t2, t3, t4 = _projections(x, wcat, bcat)
    sums = _sc_scatter(t2, edge_attr, src, dst)
    gsum, cnts = _sc_gsum(t3, t4, src, dst)

    x_out = _xside(x, x1, sums, cnts, v_gamma, v_beta)

    eb = 4000
    nb = e // eb
    be2 = be.reshape(1, d)
    acc = _edge_stats(edge_attr, gsum, We, be2, eb, nb)
    w_out = _edge_final(edge_attr, gsum, We, be2, acc, e_gamma, e_beta, eb, nb)
    return (x_out, w_out)


# speedup vs baseline: 6.5937x; 1.0002x over previous
import functools

import jax
import jax.numpy as jnp
from jax import lax
from jax.experimental import pallas as pl
from jax.experimental.pallas import tpu as pltpu
from jax.experimental.pallas import tpu_sc as plsc

NC = 2
NS = 16
NW = NC * NS
CA = 40
CB = 80
TW = 144


def _sigmoid(v):
    return 1.0 / (1.0 + jnp.exp(-v))


def _silu(v):
    return v * _sigmoid(v)


NPH = 2


def _sca_body(t2, ea, src3, dst3, zeros, sums_out,
              sidx, didx, ea0, ea1, g0, g1, ew0, ew1,
              table, sg0, sg1, se0, se1, ss0, ss1):
    n = zeros.shape[0]
    e = ea.shape[0]
    c = lax.axis_index("c")
    s = lax.axis_index("s")
    wid = c * NS + s
    rows_pt = n // NS
    ea_v = (ea0, ea1)
    g_v = (g0, g1)
    ew_v = (ew0, ew1)
    sg = (sg0, sg1)
    se = (se0, se1)
    ss = (ss0, ss1)

    pltpu.sync_copy(zeros.at[pl.ds(s * rows_pt, rows_pt)],
                    table.at[pl.ds(s * rows_pt, rows_pt)])
    plsc.subcore_barrier()

    e_pw = e // NW
    nch = e_pw // CA
    nchp = nch // NPH
    base_w = wid * e_pw

    def compute(b):
        def row(r, rc):
            for j in range(8):
                a = ea_v[b][r, pl.ds(16 * j, 16)]
                w = 1.0 / (1.0 + jnp.exp(-a))
                ew_v[b][r, pl.ds(16 * j, 16)] = w * g_v[b][r, pl.ds(16 * j, 16)]
            return rc
        lax.fori_loop(0, CA, row, 0)

    for ph in range(NPH):
        pbase = base_w + ph * nchp * CA
        pltpu.sync_copy(src3.at[wid, pl.ds(ph * nchp, nchp)], sidx)
        pltpu.sync_copy(dst3.at[wid, pl.ds(ph * nchp, nchp)], didx)

        def prefetch(k, b):
            pltpu.async_copy(ea.at[pl.ds(pbase + k * CA, CA)], ea_v[b], se[b])
            pltpu.async_copy(t2.at[didx.at[k]], g_v[b], sg[b])

        def step(k, b):
            @pl.when(k + 1 < nchp)
            def _():
                prefetch(k + 1, 1 - b)
            pltpu.make_async_copy(ea.at[pl.ds(0, CA)], ea_v[b], se[b]).wait()
            pltpu.make_async_copy(t2.at[didx.at[0]], g_v[b], sg[b]).wait()

            @pl.when(k >= 2)
            def _():
                pltpu.make_async_copy(ew_v[b], table.at[sidx.at[0]],
                                      ss[b]).wait()
            compute(b)
            pltpu.async_copy(ew_v[b], table.at[sidx.at[k]], ss[b], add=True)

        prefetch(0, 0)
        step(0, 0)

        def pair(p, carry):
            step(2 * p + 1, 1)
            step(2 * p + 2, 0)
            return carry

        lax.fori_loop(0, (nchp - 1) // 2, pair, 0)
        if nchp % 2 == 0:
            step(nchp - 1, 1)
        pltpu.make_async_copy(ew_v[0], table.at[sidx.at[0]], ss[0]).wait()
        pltpu.make_async_copy(ew_v[1], table.at[sidx.at[0]], ss[1]).wait()

    plsc.subcore_barrier()
    pltpu.sync_copy(table.at[pl.ds(s * rows_pt, rows_pt)],
                    sums_out.at[c, pl.ds(s * rows_pt, rows_pt)])


def _sc_scatter(t2, edge_attr, src, dst):
    n = t2.shape[0]
    e = edge_attr.shape[0]
    npad = ((n + 8 * NS - 1) // (8 * NS)) * (8 * NS)
    e_pw = e // NW
    nch = e_pw // CA
    nchp = nch // NPH
    src3 = src.reshape(NW, nch, CA)
    dst3 = dst.reshape(NW, nch, CA)
    zeros = jnp.zeros((npad, 128), jnp.float32)
    fn = pl.kernel(
        _sca_body,
        out_type=jax.ShapeDtypeStruct((NC, npad, 128), jnp.float32),
        mesh=plsc.VectorSubcoreMesh(core_axis_name="c", subcore_axis_name="s",
                                    num_cores=NC, num_subcores=NS),
        scratch_types=[
            pltpu.VMEM((nchp, CA), jnp.int32),
            pltpu.VMEM((nchp, CA), jnp.int32),
            pltpu.VMEM((CA, 128), jnp.float32),
            pltpu.VMEM((CA, 128), jnp.float32),
            pltpu.VMEM((CA, 128), jnp.float32),
            pltpu.VMEM((CA, 128), jnp.float32),
            pltpu.VMEM((CA, 128), jnp.float32),
            pltpu.VMEM((CA, 128), jnp.float32),
            pltpu.VMEM_SHARED((npad, 128), jnp.float32),
            pltpu.SemaphoreType.DMA,
            pltpu.SemaphoreType.DMA,
            pltpu.SemaphoreType.DMA,
            pltpu.SemaphoreType.DMA,
            pltpu.SemaphoreType.DMA,
            pltpu.SemaphoreType.DMA,
        ],
        compiler_params=pltpu.CompilerParams(use_tc_tiling_on_sc=False),
    )
    return fn(t2, edge_attr, src3, dst3, zeros)


def _scb_body(t3, t4, src3, dst3, zeros16, gsum_out, cnt_out,
              sidx, didx, g30, g31, g40, g41, gs0, gs1, ones_v,
              ctab, s30, s31, s40, s41, so0, so1, sc0, sc1):
    e = gsum_out.shape[0]
    n16 = zeros16.shape[0]
    c = lax.axis_index("c")
    s = lax.axis_index("s")
    wid = c * NS + s
    rows_pt = n16 // NS
    g3_v = (g30, g31)
    g4_v = (g40, g41)
    gs_v = (gs0, gs1)
    s3 = (s30, s31)
    s4 = (s40, s41)
    so = (so0, so1)
    sc = (sc0, sc1)

    pltpu.sync_copy(zeros16.at[pl.ds(s * rows_pt, rows_pt)],
                    ctab.at[pl.ds(s * rows_pt, rows_pt)])

    def fill(r, rc):
        ones_v[r, pl.ds(0, 16)] = jnp.full((16,), 1.0, jnp.float32)
        return rc
    lax.fori_loop(0, CB, fill, 0)
    plsc.subcore_barrier()

    e_pw = e // NW
    nch = e_pw // CB
    base_w = wid * e_pw
    pltpu.sync_copy(src3.at[wid], sidx)
    pltpu.sync_copy(dst3.at[wid], didx)

    def prefetch(k, b):
        pltpu.async_copy(t3.at[sidx.at[k]], g3_v[b], s3[b])
        pltpu.async_copy(t4.at[didx.at[k]], g4_v[b], s4[b])

    def compute(b):
        def row(r, rc):
            for j in range(8):
                gs_v[b][r, pl.ds(16 * j, 16)] = (
                    g3_v[b][r, pl.ds(16 * j, 16)]
                    + g4_v[b][r, pl.ds(16 * j, 16)])
            return rc
        lax.fori_loop(0, CB, row, 0)

    def step(k, b):
        @pl.when(k + 1 < nch)
        def _():
            prefetch(k + 1, 1 - b)
        pltpu.make_async_copy(t3.at[sidx.at[0]], g3_v[b], s3[b]).wait()
        pltpu.make_async_copy(t4.at[didx.at[0]], g4_v[b], s4[b]).wait()

        @pl.when(k >= 2)
        def _():
            pltpu.make_async_copy(gs_v[b], gsum_out.at[pl.ds(0, CB)],
                                  so[b]).wait()
            pltpu.make_async_copy(ones_v, ctab.at[sidx.at[0]], sc[b]).wait()
        compute(b)
        pltpu.async_copy(gs_v[b], gsum_out.at[pl.ds(base_w + k * CB, CB)],
                         so[b])
        pltpu.async_copy(ones_v, ctab.at[sidx.at[k]], sc[b], add=True)

    prefetch(0, 0)
    step(0, 0)

    def pair(p, carry):
        step(2 * p + 1, 1)
        step(2 * p + 2, 0)
        return carry

    lax.fori_loop(0, (nch - 1) // 2, pair, 0)
    if nch % 2 == 0:
        step(nch - 1, 1)
    for b in (0, 1):
        pltpu.make_async_copy(gs_v[b], gsum_out.at[pl.ds(0, CB)], so[b]).wait()
        pltpu.make_async_copy(ones_v, ctab.at[sidx.at[0]], sc[b]).wait()

    plsc.subcore_barrier()
    pltpu.sync_copy(ctab.at[pl.ds(s * rows_pt, rows_pt)],
                    cnt_out.at[c, pl.ds(s * rows_pt, rows_pt)])


def _sc_gsum(t3, t4, src, dst):
    e = src.shape[0]
    n = t3.shape[0]
    npad = ((n + 8 * NS - 1) // (8 * NS)) * (8 * NS)
    e_pw = e // NW
    nch = e_pw // CB
    src3 = src.reshape(NW, nch, CB)
    dst3 = dst.reshape(NW, nch, CB)
    zeros16 = jnp.zeros((npad, 16), jnp.float32)
    fn = pl.kernel(
        _scb_body,
        out_type=(jax.ShapeDtypeStruct((e, 128), jnp.float32),
                  jax.ShapeDtypeStruct((NC, npad, 16), jnp.float32)),
        mesh=plsc.VectorSubcoreMesh(core_axis_name="c", subcore_axis_name="s",
                                    num_cores=NC, num_subcores=NS),
        scratch_types=[
            pltpu.VMEM((nch, CB), jnp.int32),
            pltpu.VMEM((nch, CB), jnp.int32),
            pltpu.VMEM((CB, 128), jnp.float32),
            pltpu.VMEM((CB, 128), jnp.float32),
            pltpu.VMEM((CB, 128), jnp.float32),
            pltpu.VMEM((CB, 128), jnp.float32),
            pltpu.VMEM((CB, 128), jnp.float32),
            pltpu.VMEM((CB, 128), jnp.float32),
            pltpu.VMEM((CB, 16), jnp.float32),
            pltpu.VMEM_SHARED((npad, 16), jnp.float32),
            pltpu.SemaphoreType.DMA,
            pltpu.SemaphoreType.DMA,
            pltpu.SemaphoreType.DMA,
            pltpu.SemaphoreType.DMA,
            pltpu.SemaphoreType.DMA,
            pltpu.SemaphoreType.DMA,
            pltpu.SemaphoreType.DMA,
            pltpu.SemaphoreType.DMA,
        ],
        compiler_params=pltpu.CompilerParams(use_tc_tiling_on_sc=False),
    )
    return fn(t3, t4, src3, dst3, zeros16)


def _proj_body(x_ref, w_ref, b_ref, x1_ref, t2_ref, t3_ref, t4_ref):
    acc = jnp.dot(x_ref[...], w_ref[...],
                  preferred_element_type=jnp.float32) + b_ref[...]
    x1_ref[...] = acc[:, 0:128]
    t2_ref[...] = acc[:, 128:256]
    t3_ref[...] = acc[:, 256:384]
    t4_ref[...] = acc[:, 384:512]


def _projections(x, wcat, bcat):
    n, d = x.shape
    nb = 10
    rb = n // nb
    out = pl.pallas_call(
        _proj_body,
        grid=(nb,),
        in_specs=[
            pl.BlockSpec((rb, d), lambda i: (i, 0)),
            pl.BlockSpec((d, 4 * d), lambda i: (0, 0)),
            pl.BlockSpec((1, 4 * d), lambda i: (0, 0)),
        ],
        out_specs=[pl.BlockSpec((rb, d), lambda i: (i, 0))] * 4,
        out_shape=[jax.ShapeDtypeStruct((n, d), jnp.float32)] * 4,
    )(x, wcat, bcat)
    return out


def _xside_body(x_ref, x1_ref, sums_ref, cnts_ref, vg_ref, vb_ref, out_ref):
    n = x_ref.shape[0]
    st = sums_ref[0, 0:n, :] + sums_ref[1, 0:n, :]
    cnt = cnts_ref[0, 0:n, 0:1] + cnts_ref[1, 0:n, 0:1]
    agg = st / jnp.maximum(cnt, 1.0)
    z = x1_ref[...] + agg
    mu = jnp.mean(z, axis=0, keepdims=True)
    var = jnp.mean((z - mu) * (z - mu), axis=0, keepdims=True)
    zn = vg_ref[...] * (z - mu) * lax.rsqrt(var + 1e-5) + vb_ref[...]
    out_ref[...] = x_ref[...] + _silu(zn)


def _xside(x, x1, sums, cnts, v_gamma, v_beta):
    n, d = x.shape
    return pl.pallas_call(
        _xside_body,
        out_shape=jax.ShapeDtypeStruct((n, d), jnp.float32),
    )(x, x1, sums, cnts, v_gamma.reshape(1, d), v_beta.reshape(1, d))


def _stats_body(ea_ref, gs_ref, we_ref, be_ref, acc_ref):
    i = pl.program_id(0)

    @pl.when(i == 0)
    def _():
        acc_ref[...] = jnp.zeros_like(acc_ref)

    z = (jnp.dot(ea_ref[...], we_ref[...], preferred_element_type=jnp.float32)
         + be_ref[...] + gs_ref[...])
    s1 = jnp.sum(z, axis=0, keepdims=True)
    s2 = jnp.sum(z * z, axis=0, keepdims=True)
    pad = jnp.zeros((6, s1.shape[1]), jnp.float32)
    acc_ref[...] += jnp.concatenate([s1, s2, pad], axis=0)


def _edge_stats(edge_attr, gsum, we, be, eb, nb):
    e, d = edge_attr.shape
    return pl.pallas_call(
        _stats_body,
        grid=(nb,),
        in_specs=[
            pl.BlockSpec((eb, d), lambda i: (i, 0)),
            pl.BlockSpec((eb, d), lambda i: (i, 0)),
            pl.BlockSpec((d, d), lambda i: (0, 0)),
            pl.BlockSpec((1, d), lambda i: (0, 0)),
        ],
        out_specs=pl.BlockSpec((8, d), lambda i: (0, 0)),
        out_shape=jax.ShapeDtypeStruct((8, d), jnp.float32),
    )(edge_attr, gsum, we, be)


def _efinal_body(ea_ref, gs_ref, we_ref, be_ref, acc_ref, eg_ref, eb_ref,
                 out_ref, *, inv_e):
    ea = ea_ref[...]
    z = (jnp.dot(ea, we_ref[...], preferred_element_type=jnp.float32)
         + be_ref[...] + gs_ref[...])
    mu = acc_ref[0:1, :] * inv_e
    var = acc_ref[1:2, :] * inv_e - mu * mu
    zn = eg_ref[...] * (z - mu) * lax.rsqrt(var + 1e-5) + eb_ref[...]
    out_ref[...] = ea + _silu(zn)


def _edge_final(edge_attr, gsum, we, be, acc, e_gamma, e_beta, eb, nb):
    e, d = edge_attr.shape
    return pl.pallas_call(
        functools.partial(_efinal_body, inv_e=1.0 / e),
        grid=(nb,),
        in_specs=[
            pl.BlockSpec((eb, d), lambda i: (i, 0)),
            pl.BlockSpec((eb, d), lambda i: (i, 0)),
            pl.BlockSpec((d, d), lambda i: (0, 0)),
            pl.BlockSpec((1, d), lambda i: (0, 0)),
            pl.BlockSpec((8, d), lambda i: (0, 0)),
            pl.BlockSpec((1, d), lambda i: (0, 0)),
            pl.BlockSpec((1, d), lambda i: (0, 0)),
        ],
        out_specs=pl.BlockSpec((eb, d), lambda i: (i, 0)),
        out_shape=jax.ShapeDtypeStruct((e, d), jnp.float32),
    )(edge_attr, gsum, we, be, acc, e_gamma.reshape(1, d),
      e_beta.reshape(1, d))


def kernel(x, edge_index, edge_attr, W1, b1, W2, b2, W3, b3, W4, b4,
           We, be, v_gamma, v_beta, e_gamma, e_beta):
    n, d = x.shape
    e = edge_attr.shape[0]
    src = edge_index[0]
    dst = edge_index[1]

    wcat = jnp.concatenate([W1, W2, W3, W4], axis=1)
    bcat = jnp.concatenate([b1, b2, b3, b4]).reshape(1, 4 * d)

    x1, t2, t3, t4 = _projections(x, wcat, bcat)
    sums = _sc_scatter(t2, edge_attr, src, dst)
    gsum, cnts = _sc_gsum(t3, t4, src, dst)

    x_out = _xside(x, x1, sums, cnts, v_gamma, v_beta)

    eb = 4000
    nb = e // eb
    be2 = be.reshape(1, d)
    acc = _edge_stats(edge_attr, gsum, We, be2, eb, nb)
    w_out = _edge_final(edge_attr, gsum, We, be2, acc, e_gamma, e_beta, eb, nb)
    return (x_out, w_out)
